# baseline (speedup 1.0000x reference)
import functools
import math

import jax
import jax.numpy as jnp
from jax import lax
from jax.experimental import pallas as pl
from jax.experimental.pallas import tpu as pltpu
from jax.experimental.pallas import tpu_sc as plsc

N = 10000
E = 320000
C = 128
H = 8
D = 16

NC = 2
NS = 16
NW = NC * NS
CH = 128
NCH = 80
EP = NW * NCH * CH
CHB = 64
NCHB = EP // (NW * CHB)
GRP = 16
NGRP = CH // GRP
NBUF = 3
NSUP = 27
RPW = 624
RPW_TAIL = N - NS * RPW

BR = 400
BE_MLP = 8192
BE_SM = 16384

_INV_SQRT_D = 1.0 / math.sqrt(D)


def _qkv_body(xs_ref, xd_ref, wqt, wkt, wvt, bq, bk, bv, q_ref, kv_ref):
    xs = xs_ref[...]
    xd = xd_ref[...]
    q_ref[...] = jnp.dot(xd, wqt[...], preferred_element_type=jnp.float32) + bq[...]
    kv_ref[:, :C] = jnp.dot(xs, wkt[...], preferred_element_type=jnp.float32) + bk[...]
    kv_ref[:, C:] = jnp.dot(xs, wvt[...], preferred_element_type=jnp.float32) + bv[...]


def _qkv(x_src, x_dst, WqT, WkT, WvT, bq2, bk2, bv2):
    grid = (N // BR,)
    row_spec = pl.BlockSpec((BR, C), lambda i: (i, 0))
    w_spec = pl.BlockSpec((C, C), lambda i: (0, 0))
    b_spec = pl.BlockSpec((1, C), lambda i: (0, 0))
    return pl.pallas_call(
        _qkv_body,
        grid=grid,
        in_specs=[row_spec, row_spec, w_spec, w_spec, w_spec, b_spec, b_spec, b_spec],
        out_specs=[row_spec, pl.BlockSpec((BR, 2 * C), lambda i: (i, 0))],
        out_shape=[jax.ShapeDtypeStruct((N, C), jnp.float32),
                   jax.ShapeDtypeStruct((N, 2 * C), jnp.float32)],
    )(x_src, x_dst, WqT, WkT, WvT, bq2, bk2, bv2)


def _ebias_body(ea_ref, w1, b1, w2, b2, w3, b3, ws, bs, gp, out_ref):
    ea = ea_ref[...]
    h1 = jnp.maximum(jnp.dot(w1[...], ea, preferred_element_type=jnp.float32) + b1[...], 0.0)
    h2 = jnp.maximum(jnp.dot(w2[...], h1, preferred_element_type=jnp.float32) + b2[...], 0.0)
    mlp = jnp.dot(w3[...], h2, preferred_element_type=jnp.float32) + b3[...]
    sc = jnp.dot(ws[...], ea, preferred_element_type=jnp.float32) + bs[...]
    g = jax.nn.sigmoid(gp[0, 0])
    out_ref[...] = g * mlp + (1.0 - g) * sc


def _ebias(eaT, ew1, eb1, ew2, eb2, ew3, eb3, esw, esb, gate_param):
    ED = eaT.shape[0]
    grid = (EP // BE_MLP,)

    def full(shape):
        return pl.BlockSpec(shape, lambda i: (0, 0))

    return pl.pallas_call(
        _ebias_body,
        grid=grid,
        in_specs=[
            pl.BlockSpec((ED, BE_MLP), lambda i: (0, i)),
            full((64, ED)), full((64, 1)),
            full((32, 64)), full((32, 1)),
            full((H, 32)), full((H, 1)),
            full((H, ED)), full((H, 1)),
            full((1, 1)),
        ],
        out_specs=pl.BlockSpec((H, BE_MLP), lambda i: (0, i)),
        out_shape=jax.ShapeDtypeStruct((H, EP), jnp.float32),
    )(eaT, ew1, eb1.reshape(64, 1), ew2, eb2.reshape(32, 1), ew3, eb3.reshape(H, 1),
      esw, esb.reshape(H, 1), gate_param.reshape(1, 1))


def _dot_chunk(qrows, krows, sblk):
    iota = lax.iota(jnp.int32, GRP)

    def group(g, carry):
        rows = iota + g * GRP
        for h in range(H):
            acc = jnp.zeros((GRP,), jnp.float32)
            for d in range(D):
                col = jnp.full((GRP,), h * D + d, jnp.int32)
                qv = plsc.load_gather(qrows, [rows, col])
                kv = plsc.load_gather(krows, [rows, col])
                acc = acc + qv * kv
            plsc.store_scatter(sblk, [jnp.full((GRP,), h, jnp.int32), rows], acc)
        return carry

    lax.fori_loop(0, NGRP, group, 0)


def _scores_sc_body(q_hbm, kv_hbm, src_hbm, dst_hbm, s_hbm, ve_hbm,
                    asrc, adst, q0, q1, kv0, kv1, s0, s1,
                    si0, si1, si2, si3, sg0, sg1, sw0, sw1, sv0, sv1):
    cid = lax.axis_index("c")
    sid = lax.axis_index("s")
    wid = cid * NS + sid
    qrows = [q0, q1]
    kvrows = [kv0, kv1]
    sblk = [s0, s1]
    sem_i = [si0, si1, si2, si3]
    sem_g = [sg0, sg1]
    sem_w = [sw0, sw1]
    sem_v = [sv0, sv1]
    cbase = wid * NCH * CH

    def issue_idx(j, b4):
        base = cbase + j * CH
        pltpu.async_copy(src_hbm.at[pl.ds(base, CH)], asrc.at[b4], sem_i[b4])
        pltpu.async_copy(dst_hbm.at[pl.ds(base, CH)], adst.at[b4], sem_i[b4])

    def wait_idx(b4):
        pltpu.make_async_copy(src_hbm.at[pl.ds(0, CH)], asrc.at[b4],
                              sem_i[b4]).wait()
        pltpu.make_async_copy(dst_hbm.at[pl.ds(0, CH)], adst.at[b4],
                              sem_i[b4]).wait()

    def issue_gather(b2, b4):
        pltpu.async_copy(kv_hbm.at[asrc.at[b4]], kvrows[b2], sem_g[b2])
        pltpu.async_copy(q_hbm.at[adst.at[b4]], qrows[b2], sem_g[b2])

    def wait_gather(b2, b4):
        pltpu.make_async_copy(kv_hbm.at[asrc.at[b4]], kvrows[b2],
                              sem_g[b2]).wait()
        pltpu.make_async_copy(q_hbm.at[adst.at[b4]], qrows[b2],
                              sem_g[b2]).wait()

    def wait_vwrite(b2):
        pltpu.make_async_copy(kvrows[b2].at[:, pl.ds(C, C)],
                              ve_hbm.at[pl.ds(0, CH)], sem_v[b2]).wait()

    issue_idx(0, 0)
    issue_idx(1, 1)
    wait_idx(0)
    issue_gather(0, 0)

    def super_iter(jj, carry):
        for u in range(4):
            j = jj * 4 + u
            b2 = u % 2
            nb2 = (u + 1) % 2

            @pl.when(j + 2 < NCH)
            def _():
                issue_idx(j + 2, (u + 2) % 4)

            @pl.when(j + 1 < NCH)
            def _():
                wait_idx((u + 1) % 4)

                @pl.when(j >= 1)
                def _():
                    wait_vwrite(nb2)

                issue_gather(nb2, (u + 1) % 4)

            @pl.when(j >= 2)
            def _():
                pltpu.make_async_copy(
                    sblk[b2], s_hbm.at[:, pl.ds(0, CH)], sem_w[b2]).wait()

            wait_gather(b2, u)
            pltpu.async_copy(kvrows[b2].at[:, pl.ds(C, C)],
                             ve_hbm.at[pl.ds(cbase + j * CH, CH)], sem_v[b2])
            _dot_chunk(qrows[b2], kvrows[b2], sblk[b2])
            pltpu.async_copy(
                sblk[b2], s_hbm.at[:, pl.ds(cbase + j * CH, CH)], sem_w[b2])
        return carry

    lax.fori_loop(0, NCH // 4, super_iter, 0)
    for b2 in range(2):
        pltpu.make_async_copy(sblk[b2], s_hbm.at[:, pl.ds(0, CH)],
                              sem_w[b2]).wait()
        wait_vwrite(b2)


def _scores_sc(Q, KV, srcp, dstp):
    mesh = plsc.VectorSubcoreMesh(core_axis_name="c", subcore_axis_name="s",
                                  num_cores=NC, num_subcores=NS)
    rowbuf = pltpu.VMEM((CH, C), jnp.float32)
    kvbuf = pltpu.VMEM((CH, 2 * C), jnp.float32)
    f = functools.partial(
        pl.kernel,
        out_type=[jax.ShapeDtypeStruct((H, EP), jnp.float32),
                  jax.ShapeDtypeStruct((EP, C), jnp.float32)],
        mesh=mesh,
        scratch_types=[
            pltpu.VMEM((4, CH), jnp.int32),
            pltpu.VMEM((4, CH), jnp.int32),
            rowbuf, rowbuf,
            kvbuf, kvbuf,
            pltpu.VMEM((H, CH), jnp.float32),
            pltpu.VMEM((H, CH), jnp.float32),
            pltpu.SemaphoreType.DMA, pltpu.SemaphoreType.DMA,
            pltpu.SemaphoreType.DMA, pltpu.SemaphoreType.DMA,
            pltpu.SemaphoreType.DMA, pltpu.SemaphoreType.DMA,
            pltpu.SemaphoreType.DMA, pltpu.SemaphoreType.DMA,
            pltpu.SemaphoreType.DMA, pltpu.SemaphoreType.DMA,
        ],
        compiler_params=pltpu.CompilerParams(needs_layout_passes=False),
    )(_scores_sc_body)
    return f(Q, KV, srcp, dstp)


def _softmax_body(s_ref, b_ref, a_ref, macc, sacc):
    p = pl.program_id(0)
    j = pl.program_id(1)
    s = s_ref[...] * _INV_SQRT_D + b_ref[...]
    col = lax.broadcasted_iota(jnp.int32, (H, BE_SM), 1) + j * BE_SM
    s = jnp.where(col < E, s, -1e30)

    @pl.when(jnp.logical_and(p == 0, j == 0))
    def _():
        macc[...] = jnp.full((H, 128), -1e30, jnp.float32)

    @pl.when(p == 0)
    def _():
        m = jnp.max(s, axis=1, keepdims=True)
        macc[...] = jnp.maximum(macc[...], jnp.broadcast_to(m, (H, 128)))
        a_ref[...] = s

    @pl.when(jnp.logical_and(p == 1, j == 0))
    def _():
        sacc[...] = jnp.zeros((H, 128), jnp.float32)

    @pl.when(p == 1)
    def _():
        ex = jnp.exp(s - macc[:, 0:1])
        sacc[...] += jnp.broadcast_to(jnp.sum(ex, axis=1, keepdims=True), (H, 128))
        a_ref[...] = s

    @pl.when(p == 2)
    def _():
        a_ref[...] = jnp.exp(s - macc[:, 0:1]) / sacc[:, 0:1]


def _softmax(scores, bias):
    grid = (3, EP // BE_SM)
    spec = pl.BlockSpec((H, BE_SM), lambda p, j: (0, j))
    return pl.pallas_call(
        _softmax_body,
        grid=grid,
        in_specs=[spec, spec],
        out_specs=spec,
        out_shape=jax.ShapeDtypeStruct((H, EP), jnp.float32),
        scratch_shapes=[
            pltpu.VMEM((H, 128), jnp.float32),
            pltpu.VMEM((H, 128), jnp.float32),
        ],
    )(scores, bias)


def _scale_chunk(vrows, ablk):
    iota = lax.iota(jnp.int32, GRP)
    hrow = [jnp.full((GRP,), h, jnp.int32) for h in range(H)]

    def edge(e, carry):
        erow = jnp.full((GRP,), e, jnp.int32)
        for h in range(H):
            cols = iota + h * D
            av = plsc.load_gather(ablk, [hrow[h // 2], erow + (h % 2) * CHB])
            vv = plsc.load_gather(vrows, [erow, cols])
            plsc.store_scatter(vrows, [erow, cols], vv * av)
        return carry

    lax.fori_loop(0, CHB, edge, 0)


def _agg_sc_body(ve_hbm, attn_hbm, dst_hbm, zeros_hbm, out_hbm,
                 adst, v0, v1, v2, v3, a0, a1, a2, a3, shared,
                 si0, si1, si2, si3, si4, si5, si6, si7,
                 sg0, sg1, sg2, sg3, ss0, ss1, ss2, ss3):
    cid = lax.axis_index("c")
    sid = lax.axis_index("s")
    wid = cid * NS + sid
    vrows = [v0, v1, v2, v3]
    ablk = [a0, a1, a2, a3]
    sem_i = [si0, si1, si2, si3, si4, si5, si6, si7]
    sem_g = [sg0, sg1, sg2, sg3]
    sem_s = [ss0, ss1, ss2, ss3]
    cbase = wid * NCHB * CHB

    pltpu.sync_copy(zeros_hbm.at[pl.ds(sid * RPW, RPW)],
                    shared.at[pl.ds(sid * RPW, RPW)])

    @pl.when(sid == NS - 1)
    def _():
        pltpu.sync_copy(zeros_hbm.at[pl.ds(NS * RPW, RPW_TAIL)],
                        shared.at[pl.ds(NS * RPW, RPW_TAIL)])

    def issue_idx(j, b8):
        base = cbase + j * CHB
        pltpu.async_copy(dst_hbm.at[pl.ds(base, CHB)], adst.at[b8], sem_i[b8])

    def wait_idx(b8):
        pltpu.make_async_copy(dst_hbm.at[pl.ds(0, CHB)], adst.at[b8],
                              sem_i[b8]).wait()

    def issue_gather(j, b4, b8):
        pltpu.async_copy(ve_hbm.at[pl.ds(cbase + j * CHB, CHB)], vrows[b4],
                         sem_g[b4])
        pltpu.async_copy(attn_hbm.at[wid * NCHB + j], ablk[b4], sem_g[b4])

    def wait_gather(b4, b8):
        pltpu.make_async_copy(ve_hbm.at[pl.ds(0, CHB)], vrows[b4],
                              sem_g[b4]).wait()
        pltpu.make_async_copy(attn_hbm.at[0], ablk[b4], sem_g[b4]).wait()

    def wait_scatter(b4):
        pltpu.make_async_copy(vrows[b4], shared.at[adst.at[0]],
                              sem_s[b4]).wait()

    issue_idx(0, 0)
    issue_idx(1, 1)
    issue_idx(2, 2)
    wait_idx(0)
    issue_gather(0, 0, 0)
    plsc.subcore_barrier()

    def super_iter(jj, carry):
        for u in range(8):
            j = jj * 8 + u
            b4 = u % 4

            @pl.when(j + 3 < NCHB)
            def _():
                issue_idx(j + 3, (u + 3) % 8)

            @pl.when(j + 1 < NCHB)
            def _():
                wait_idx((u + 1) % 8)

                @pl.when(j + 1 >= 4)
                def _():
                    wait_scatter((u + 1) % 4)

                issue_gather(j + 1, (u + 1) % 4, (u + 1) % 8)

            wait_gather(b4, u)
            _scale_chunk(vrows[b4], ablk[b4])
            pltpu.async_copy(vrows[b4], shared.at[adst.at[u]], sem_s[b4],
                             add=True)
        return carry

    lax.fori_loop(0, NCHB // 8, super_iter, 0)
    for b4 in range(4):
        wait_scatter(b4)
    plsc.subcore_barrier()
    pltpu.sync_copy(shared.at[pl.ds(sid * RPW, RPW)],
                    out_hbm.at[cid, pl.ds(sid * RPW, RPW)])

    @pl.when(sid == NS - 1)
    def _():
        pltpu.sync_copy(shared.at[pl.ds(NS * RPW, RPW_TAIL)],
                        out_hbm.at[cid, pl.ds(NS * RPW, RPW_TAIL)])


def _agg_sc(Ve, attn, dstp):
    mesh = plsc.VectorSubcoreMesh(core_axis_name="c", subcore_axis_name="s",
                                  num_cores=NC, num_subcores=NS)
    zeros = jnp.zeros((N, C), jnp.float32)
    rowbuf = pltpu.VMEM((CHB, C), jnp.float32)
    abuf = pltpu.VMEM(((H * CHB) // 128, 128), jnp.float32)
    f = functools.partial(
        pl.kernel,
        out_type=jax.ShapeDtypeStruct((NC, N, C), jnp.float32),
        mesh=mesh,
        scratch_types=[
            pltpu.VMEM((8, CHB), jnp.int32),
            rowbuf, rowbuf, rowbuf, rowbuf,
            abuf, abuf, abuf, abuf,
            pltpu.VMEM_SHARED((N, C), jnp.float32),
            pltpu.SemaphoreType.DMA, pltpu.SemaphoreType.DMA,
            pltpu.SemaphoreType.DMA, pltpu.SemaphoreType.DMA,
            pltpu.SemaphoreType.DMA, pltpu.SemaphoreType.DMA,
            pltpu.SemaphoreType.DMA, pltpu.SemaphoreType.DMA,
            pltpu.SemaphoreType.DMA, pltpu.SemaphoreType.DMA,
            pltpu.SemaphoreType.DMA, pltpu.SemaphoreType.DMA,
            pltpu.SemaphoreType.DMA, pltpu.SemaphoreType.DMA,
            pltpu.SemaphoreType.DMA, pltpu.SemaphoreType.DMA,
        ],
        compiler_params=pltpu.CompilerParams(needs_layout_passes=False),
    )(_agg_sc_body)
    return f(Ve, attn, dstp, zeros)


def _omlp_body(p0_ref, p1_ref, xd_ref, w1t, b1, w2t, b2, w3t, b3,
               pos, gamma, beta, o_ref):
    agg = p0_ref[...] + p1_ref[...]
    o1 = jnp.maximum(jnp.dot(agg, w1t[...], preferred_element_type=jnp.float32) + b1[...], 0.0)
    o2 = jnp.maximum(jnp.dot(o1, w2t[...], preferred_element_type=jnp.float32) + b2[...], 0.0)
    o3 = jnp.dot(o2, w3t[...], preferred_element_type=jnp.float32) + b3[...]
    o3 = o3 + xd_ref[...] + pos[...]
    mu = jnp.mean(o3, axis=1, keepdims=True)
    ctr = o3 - mu
    var = jnp.mean(ctr * ctr, axis=1, keepdims=True)
    o_ref[...] = ctr * lax.rsqrt(var + 1e-5) * gamma[...] + beta[...]


def _omlp(p0, p1, x_dst, ow1T, ob1, ow2T, ob2, ow3T, ob3, pos, gamma, beta):
    grid = (N // BR,)
    row_spec = pl.BlockSpec((BR, C), lambda i: (i, 0))

    def full(shape):
        return pl.BlockSpec(shape, lambda i: (0, 0))

    return pl.pallas_call(
        _omlp_body,
        grid=grid,
        in_specs=[
            row_spec, row_spec, row_spec,
            full((C, 2 * C)), full((1, 2 * C)),
            full((2 * C, 3 * C)), full((1, 3 * C)),
            full((3 * C, C)), full((1, C)),
            full((1, C)), full((1, C)), full((1, C)),
        ],
        out_specs=row_spec,
        out_shape=jax.ShapeDtypeStruct((N, C), jnp.float32),
    )(p0, p1, x_dst, ow1T, ob1.reshape(1, 2 * C), ow2T, ob2.reshape(1, 3 * C),
      ow3T, ob3.reshape(1, C), pos, gamma.reshape(1, C), beta.reshape(1, C))


def kernel(x_src, x_dst, edge_attr, Wq, bq, Wk, bk, Wv, bv,
           ew1, eb1, ew2, eb2, ew3, eb3, esw, esb, gate_param,
           ow1, ob1, ow2, ob2, ow3, ob3, pos, gamma, beta, edge_index):
    pad = jnp.zeros((EP - E,), jnp.int32)
    srcp = jnp.concatenate([edge_index[0], pad])
    dstp = jnp.concatenate([edge_index[1], pad])
    eaT = jnp.concatenate(
        [edge_attr, jnp.zeros((EP - E, edge_attr.shape[1]), jnp.float32)]).T

    Q, KV = _qkv(x_src, x_dst, Wq.T, Wk.T, Wv.T,
                 bq.reshape(1, C), bk.reshape(1, C), bv.reshape(1, C))
    bias = _ebias(eaT, ew1, eb1, ew2, eb2, ew3, eb3, esw, esb, gate_param)
    scores, Ve = _scores_sc(Q, KV, srcp, dstp)
    attn = _softmax(scores, bias)
    attn_c = attn.reshape(H, EP // CHB, CHB).transpose(1, 0, 2).reshape(
        EP // CHB, (H * CHB) // 128, 128)
    parts = _agg_sc(Ve, attn_c, dstp)
    out = _omlp(parts[0], parts[1], x_dst, ow1.T, ob1, ow2.T, ob2, ow3.T, ob3,
                pos, gamma, beta)
    return out


# speedup vs baseline: 1.5403x; 1.5403x over previous
import functools
import math

import jax
import jax.numpy as jnp
from jax import lax
from jax.experimental import pallas as pl
from jax.experimental.pallas import tpu as pltpu
from jax.experimental.pallas import tpu_sc as plsc

N = 10000
E = 320000
C = 128
H = 8
D = 16

NC = 2
NS = 16
NW = NC * NS
CH = 128
NCH = 80
EP = NW * NCH * CH
CHB = 64
NCHB = EP // (NW * CHB)
GRP = 16
NGRP = CH // GRP
NBUF = 3
NSUP = 27
RPW = 624
RPW_TAIL = N - NS * RPW

BR = 400
BE_MLP = 8192
BE_SM = 16384

_INV_SQRT_D = 1.0 / math.sqrt(D)


def _qkv_body(xs_ref, xd_ref, wqt, wkt, wvt, bq, bk, bv, q_ref, k_ref, v_ref):
    xs = xs_ref[...]
    xd = xd_ref[...]
    q_ref[...] = jnp.dot(xd, wqt[...], preferred_element_type=jnp.float32) + bq[...]
    k_ref[...] = jnp.dot(xs, wkt[...], preferred_element_type=jnp.float32) + bk[...]
    v_ref[...] = jnp.dot(xs, wvt[...], preferred_element_type=jnp.float32) + bv[...]


def _qkv(x_src, x_dst, WqT, WkT, WvT, bq2, bk2, bv2):
    grid = (N // BR,)
    row_spec = pl.BlockSpec((BR, C), lambda i: (i, 0))
    w_spec = pl.BlockSpec((C, C), lambda i: (0, 0))
    b_spec = pl.BlockSpec((1, C), lambda i: (0, 0))
    out = jax.ShapeDtypeStruct((N, C), jnp.float32)
    return pl.pallas_call(
        _qkv_body,
        grid=grid,
        in_specs=[row_spec, row_spec, w_spec, w_spec, w_spec, b_spec, b_spec, b_spec],
        out_specs=[row_spec, row_spec, row_spec],
        out_shape=[out, out, out],
    )(x_src, x_dst, WqT, WkT, WvT, bq2, bk2, bv2)


def _ebias_body(ea_ref, w1, b1, w2, b2, w3, b3, ws, bs, gp, out_ref):
    ea = ea_ref[...]
    h1 = jnp.maximum(jnp.dot(w1[...], ea, preferred_element_type=jnp.float32) + b1[...], 0.0)
    h2 = jnp.maximum(jnp.dot(w2[...], h1, preferred_element_type=jnp.float32) + b2[...], 0.0)
    mlp = jnp.dot(w3[...], h2, preferred_element_type=jnp.float32) + b3[...]
    sc = jnp.dot(ws[...], ea, preferred_element_type=jnp.float32) + bs[...]
    g = jax.nn.sigmoid(gp[0, 0])
    out_ref[...] = g * mlp + (1.0 - g) * sc


def _ebias(eaT, ew1, eb1, ew2, eb2, ew3, eb3, esw, esb, gate_param):
    ED = eaT.shape[0]
    grid = (EP // BE_MLP,)

    def full(shape):
        return pl.BlockSpec(shape, lambda i: (0, 0))

    return pl.pallas_call(
        _ebias_body,
        grid=grid,
        in_specs=[
            pl.BlockSpec((ED, BE_MLP), lambda i: (0, i)),
            full((64, ED)), full((64, 1)),
            full((32, 64)), full((32, 1)),
            full((H, 32)), full((H, 1)),
            full((H, ED)), full((H, 1)),
            full((1, 1)),
        ],
        out_specs=pl.BlockSpec((H, BE_MLP), lambda i: (0, i)),
        out_shape=jax.ShapeDtypeStruct((H, EP), jnp.float32),
    )(eaT, ew1, eb1.reshape(64, 1), ew2, eb2.reshape(32, 1), ew3, eb3.reshape(H, 1),
      esw, esb.reshape(H, 1), gate_param.reshape(1, 1))


def _dot_chunk(qrows, krows, sblk):
    iota = lax.iota(jnp.int32, GRP)

    def group(g, carry):
        rows = iota + g * GRP
        for h in range(H):
            acc = jnp.zeros((GRP,), jnp.float32)
            for d in range(D):
                col = jnp.full((GRP,), h * D + d, jnp.int32)
                qv = plsc.load_gather(qrows, [rows, col])
                kv = plsc.load_gather(krows, [rows, col])
                acc = acc + qv * kv
            plsc.store_scatter(sblk, [jnp.full((GRP,), h, jnp.int32), rows], acc)
        return carry

    lax.fori_loop(0, NGRP, group, 0)


def _scores_sc_body(q_hbm, k_hbm, src_hbm, dst_hbm, s_hbm,
                    asrc, adst, q0, q1, k0, k1, s0, s1,
                    si0, si1, si2, si3, sg0, sg1, sw0, sw1):
    cid = lax.axis_index("c")
    sid = lax.axis_index("s")
    wid = cid * NS + sid
    qrows = [q0, q1]
    krows = [k0, k1]
    sblk = [s0, s1]
    sem_i = [si0, si1, si2, si3]
    sem_g = [sg0, sg1]
    sem_w = [sw0, sw1]
    cbase = wid * NCH * CH

    def issue_idx(j, b4):
        base = cbase + j * CH
        pltpu.async_copy(src_hbm.at[pl.ds(base, CH)], asrc.at[b4], sem_i[b4])
        pltpu.async_copy(dst_hbm.at[pl.ds(base, CH)], adst.at[b4], sem_i[b4])

    def wait_idx(b4):
        pltpu.make_async_copy(src_hbm.at[pl.ds(0, CH)], asrc.at[b4],
                              sem_i[b4]).wait()
        pltpu.make_async_copy(dst_hbm.at[pl.ds(0, CH)], adst.at[b4],
                              sem_i[b4]).wait()

    def issue_gather(b2, b4):
        pltpu.async_copy(k_hbm.at[asrc.at[b4]], krows[b2], sem_g[b2])
        pltpu.async_copy(q_hbm.at[adst.at[b4]], qrows[b2], sem_g[b2])

    def wait_gather(b2, b4):
        pltpu.make_async_copy(k_hbm.at[asrc.at[b4]], krows[b2],
                              sem_g[b2]).wait()
        pltpu.make_async_copy(q_hbm.at[adst.at[b4]], qrows[b2],
                              sem_g[b2]).wait()

    issue_idx(0, 0)
    issue_idx(1, 1)
    wait_idx(0)
    issue_gather(0, 0)

    def super_iter(jj, carry):
        for u in range(4):
            j = jj * 4 + u
            b2 = u % 2
            nb2 = (u + 1) % 2

            @pl.when(j + 2 < NCH)
            def _():
                issue_idx(j + 2, (u + 2) % 4)

            @pl.when(j + 1 < NCH)
            def _():
                wait_idx((u + 1) % 4)
                issue_gather(nb2, (u + 1) % 4)

            @pl.when(j >= 2)
            def _():
                pltpu.make_async_copy(
                    sblk[b2], s_hbm.at[:, pl.ds(0, CH)], sem_w[b2]).wait()

            wait_gather(b2, u)
            pltpu.async_copy(
                sblk[b2], s_hbm.at[:, pl.ds(cbase + j * CH, CH)], sem_w[b2])
        return carry

    lax.fori_loop(0, NCH // 4, super_iter, 0)
    for b2 in range(2):
        pltpu.make_async_copy(sblk[b2], s_hbm.at[:, pl.ds(0, CH)],
                              sem_w[b2]).wait()


def _scores_sc(Q, K, srcp, dstp):
    mesh = plsc.VectorSubcoreMesh(core_axis_name="c", subcore_axis_name="s",
                                  num_cores=NC, num_subcores=NS)
    rowbuf = pltpu.VMEM((CH, C), jnp.float32)
    f = functools.partial(
        pl.kernel,
        out_type=jax.ShapeDtypeStruct((H, EP), jnp.float32),
        mesh=mesh,
        scratch_types=[
            pltpu.VMEM((4, CH), jnp.int32),
            pltpu.VMEM((4, CH), jnp.int32),
            rowbuf, rowbuf,
            rowbuf, rowbuf,
            pltpu.VMEM((H, CH), jnp.float32),
            pltpu.VMEM((H, CH), jnp.float32),
            pltpu.SemaphoreType.DMA, pltpu.SemaphoreType.DMA,
            pltpu.SemaphoreType.DMA, pltpu.SemaphoreType.DMA,
            pltpu.SemaphoreType.DMA, pltpu.SemaphoreType.DMA,
            pltpu.SemaphoreType.DMA, pltpu.SemaphoreType.DMA,
        ],
        compiler_params=pltpu.CompilerParams(needs_layout_passes=False),
    )(_scores_sc_body)
    return f(Q, K, srcp, dstp)


def _softmax_body(s_ref, b_ref, a_ref, macc, sacc):
    p = pl.program_id(0)
    j = pl.program_id(1)
    s = s_ref[...] * _INV_SQRT_D + b_ref[...]
    col = lax.broadcasted_iota(jnp.int32, (H, BE_SM), 1) + j * BE_SM
    s = jnp.where(col < E, s, -1e30)

    @pl.when(jnp.logical_and(p == 0, j == 0))
    def _():
        macc[...] = jnp.full((H, 128), -1e30, jnp.float32)

    @pl.when(p == 0)
    def _():
        m = jnp.max(s, axis=1, keepdims=True)
        macc[...] = jnp.maximum(macc[...], jnp.broadcast_to(m, (H, 128)))
        a_ref[...] = s

    @pl.when(jnp.logical_and(p == 1, j == 0))
    def _():
        sacc[...] = jnp.zeros((H, 128), jnp.float32)

    @pl.when(p == 1)
    def _():
        ex = jnp.exp(s - macc[:, 0:1])
        sacc[...] += jnp.broadcast_to(jnp.sum(ex, axis=1, keepdims=True), (H, 128))
        a_ref[...] = s

    @pl.when(p == 2)
    def _():
        a_ref[...] = jnp.exp(s - macc[:, 0:1]) / sacc[:, 0:1]


def _softmax(scores, bias):
    grid = (3, EP // BE_SM)
    spec = pl.BlockSpec((H, BE_SM), lambda p, j: (0, j))
    return pl.pallas_call(
        _softmax_body,
        grid=grid,
        in_specs=[spec, spec],
        out_specs=spec,
        out_shape=jax.ShapeDtypeStruct((H, EP), jnp.float32),
        scratch_shapes=[
            pltpu.VMEM((H, 128), jnp.float32),
            pltpu.VMEM((H, 128), jnp.float32),
        ],
    )(scores, bias)


def _scale_chunk(vrows, ablk):
    iota = lax.iota(jnp.int32, GRP)
    hrow = [jnp.full((GRP,), h, jnp.int32) for h in range(H)]

    def edge(e, carry):
        erow = jnp.full((GRP,), e, jnp.int32)
        for h in range(H):
            cols = iota + h * D
            av = plsc.load_gather(ablk, [hrow[h // 2], erow + (h % 2) * CHB])
            vv = plsc.load_gather(vrows, [erow, cols])
            plsc.store_scatter(vrows, [erow, cols], vv * av)
        return carry

    lax.fori_loop(0, CHB, edge, 0)


def _agg_sc_body(v_hbm, attn_hbm, src_hbm, dst_hbm, zeros_hbm, out_hbm,
                 asrc, adst, v0, v1, v2, v3, a0, a1, a2, a3, shared,
                 si0, si1, si2, si3, si4, si5, si6, si7,
                 sg0, sg1, sg2, sg3, ss0, ss1, ss2, ss3):
    cid = lax.axis_index("c")
    sid = lax.axis_index("s")
    wid = cid * NS + sid
    vrows = [v0, v1, v2, v3]
    ablk = [a0, a1, a2, a3]
    sem_i = [si0, si1, si2, si3, si4, si5, si6, si7]
    sem_g = [sg0, sg1, sg2, sg3]
    sem_s = [ss0, ss1, ss2, ss3]
    cbase = wid * NCHB * CHB

    pltpu.sync_copy(zeros_hbm.at[pl.ds(sid * RPW, RPW)],
                    shared.at[pl.ds(sid * RPW, RPW)])

    @pl.when(sid == NS - 1)
    def _():
        pltpu.sync_copy(zeros_hbm.at[pl.ds(NS * RPW, RPW_TAIL)],
                        shared.at[pl.ds(NS * RPW, RPW_TAIL)])

    def issue_idx(j, b8):
        base = cbase + j * CHB
        pltpu.async_copy(src_hbm.at[pl.ds(base, CHB)], asrc.at[b8], sem_i[b8])
        pltpu.async_copy(dst_hbm.at[pl.ds(base, CHB)], adst.at[b8], sem_i[b8])

    def wait_idx(b8):
        pltpu.make_async_copy(src_hbm.at[pl.ds(0, CHB)], asrc.at[b8],
                              sem_i[b8]).wait()
        pltpu.make_async_copy(dst_hbm.at[pl.ds(0, CHB)], adst.at[b8],
                              sem_i[b8]).wait()

    def issue_gather(j, b4, b8):
        pltpu.async_copy(v_hbm.at[asrc.at[b8]], vrows[b4], sem_g[b4])
        pltpu.async_copy(attn_hbm.at[wid * NCHB + j], ablk[b4], sem_g[b4])

    def wait_gather(b4, b8):
        pltpu.make_async_copy(v_hbm.at[asrc.at[b8]], vrows[b4],
                              sem_g[b4]).wait()
        pltpu.make_async_copy(attn_hbm.at[0], ablk[b4], sem_g[b4]).wait()

    def wait_scatter(b4):
        pltpu.make_async_copy(vrows[b4], shared.at[adst.at[0]],
                              sem_s[b4]).wait()

    issue_idx(0, 0)
    issue_idx(1, 1)
    issue_idx(2, 2)
    wait_idx(0)
    issue_gather(0, 0, 0)
    plsc.subcore_barrier()

    def super_iter(jj, carry):
        for u in range(8):
            j = jj * 8 + u
            b4 = u % 4

            @pl.when(j + 3 < NCHB)
            def _():
                issue_idx(j + 3, (u + 3) % 8)

            @pl.when(j + 1 < NCHB)
            def _():
                wait_idx((u + 1) % 8)

                @pl.when(j + 1 >= 4)
                def _():
                    wait_scatter((u + 1) % 4)

                issue_gather(j + 1, (u + 1) % 4, (u + 1) % 8)

            wait_gather(b4, u)
            _scale_chunk(vrows[b4], ablk[b4])
            pltpu.async_copy(vrows[b4], shared.at[adst.at[u]], sem_s[b4],
                             add=True)
        return carry

    lax.fori_loop(0, NCHB // 8, super_iter, 0)
    for b4 in range(4):
        wait_scatter(b4)
    plsc.subcore_barrier()
    pltpu.sync_copy(shared.at[pl.ds(sid * RPW, RPW)],
                    out_hbm.at[cid, pl.ds(sid * RPW, RPW)])

    @pl.when(sid == NS - 1)
    def _():
        pltpu.sync_copy(shared.at[pl.ds(NS * RPW, RPW_TAIL)],
                        out_hbm.at[cid, pl.ds(NS * RPW, RPW_TAIL)])


def _agg_sc(V, attn, srcp, dstp):
    mesh = plsc.VectorSubcoreMesh(core_axis_name="c", subcore_axis_name="s",
                                  num_cores=NC, num_subcores=NS)
    zeros = jnp.zeros((N, C), jnp.float32)
    rowbuf = pltpu.VMEM((CHB, C), jnp.float32)
    abuf = pltpu.VMEM(((H * CHB) // 128, 128), jnp.float32)
    f = functools.partial(
        pl.kernel,
        out_type=jax.ShapeDtypeStruct((NC, N, C), jnp.float32),
        mesh=mesh,
        scratch_types=[
            pltpu.VMEM((8, CHB), jnp.int32),
            pltpu.VMEM((8, CHB), jnp.int32),
            rowbuf, rowbuf, rowbuf, rowbuf,
            abuf, abuf, abuf, abuf,
            pltpu.VMEM_SHARED((N, C), jnp.float32),
            pltpu.SemaphoreType.DMA, pltpu.SemaphoreType.DMA,
            pltpu.SemaphoreType.DMA, pltpu.SemaphoreType.DMA,
            pltpu.SemaphoreType.DMA, pltpu.SemaphoreType.DMA,
            pltpu.SemaphoreType.DMA, pltpu.SemaphoreType.DMA,
            pltpu.SemaphoreType.DMA, pltpu.SemaphoreType.DMA,
            pltpu.SemaphoreType.DMA, pltpu.SemaphoreType.DMA,
            pltpu.SemaphoreType.DMA, pltpu.SemaphoreType.DMA,
            pltpu.SemaphoreType.DMA, pltpu.SemaphoreType.DMA,
        ],
        compiler_params=pltpu.CompilerParams(needs_layout_passes=False),
    )(_agg_sc_body)
    return f(V, attn, srcp, dstp, zeros)


def _omlp_body(p0_ref, p1_ref, xd_ref, w1t, b1, w2t, b2, w3t, b3,
               pos, gamma, beta, o_ref):
    agg = p0_ref[...] + p1_ref[...]
    o1 = jnp.maximum(jnp.dot(agg, w1t[...], preferred_element_type=jnp.float32) + b1[...], 0.0)
    o2 = jnp.maximum(jnp.dot(o1, w2t[...], preferred_element_type=jnp.float32) + b2[...], 0.0)
    o3 = jnp.dot(o2, w3t[...], preferred_element_type=jnp.float32) + b3[...]
    o3 = o3 + xd_ref[...] + pos[...]
    mu = jnp.mean(o3, axis=1, keepdims=True)
    ctr = o3 - mu
    var = jnp.mean(ctr * ctr, axis=1, keepdims=True)
    o_ref[...] = ctr * lax.rsqrt(var + 1e-5) * gamma[...] + beta[...]


def _omlp(p0, p1, x_dst, ow1T, ob1, ow2T, ob2, ow3T, ob3, pos, gamma, beta):
    grid = (N // BR,)
    row_spec = pl.BlockSpec((BR, C), lambda i: (i, 0))

    def full(shape):
        return pl.BlockSpec(shape, lambda i: (0, 0))

    return pl.pallas_call(
        _omlp_body,
        grid=grid,
        in_specs=[
            row_spec, row_spec, row_spec,
            full((C, 2 * C)), full((1, 2 * C)),
            full((2 * C, 3 * C)), full((1, 3 * C)),
            full((3 * C, C)), full((1, C)),
            full((1, C)), full((1, C)), full((1, C)),
        ],
        out_specs=row_spec,
        out_shape=jax.ShapeDtypeStruct((N, C), jnp.float32),
    )(p0, p1, x_dst, ow1T, ob1.reshape(1, 2 * C), ow2T, ob2.reshape(1, 3 * C),
      ow3T, ob3.reshape(1, C), pos, gamma.reshape(1, C), beta.reshape(1, C))


def kernel(x_src, x_dst, edge_attr, Wq, bq, Wk, bk, Wv, bv,
           ew1, eb1, ew2, eb2, ew3, eb3, esw, esb, gate_param,
           ow1, ob1, ow2, ob2, ow3, ob3, pos, gamma, beta, edge_index):
    pad = jnp.zeros((EP - E,), jnp.int32)
    srcp = jnp.concatenate([edge_index[0], pad])
    dstp = jnp.concatenate([edge_index[1], pad])
    eaT = jnp.concatenate(
        [edge_attr, jnp.zeros((EP - E, edge_attr.shape[1]), jnp.float32)]).T

    Q, K, V = _qkv(x_src, x_dst, Wq.T, Wk.T, Wv.T,
                   bq.reshape(1, C), bk.reshape(1, C), bv.reshape(1, C))
    bias = _ebias(eaT, ew1, eb1, ew2, eb2, ew3, eb3, esw, esb, gate_param)
    scores = _scores_sc(Q, K, srcp, dstp)
    attn = _softmax(scores, bias)
    attn_c = attn.reshape(H, EP // CHB, CHB).transpose(1, 0, 2).reshape(
        EP // CHB, (H * CHB) // 128, 128)
    parts = _agg_sc(V, attn_c, srcp, dstp)
    out = _omlp(parts[0], parts[1], x_dst, ow1.T, ob1, ow2.T, ob2, ow3.T, ob3,
                pos, gamma, beta)
    return out
